# baseline (speedup 1.0000x reference)
import functools

import jax
import jax.numpy as jnp
from jax import lax
from jax.experimental import pallas as pl
from jax.experimental.pallas import tpu as pltpu

_XI = 10.0
_EPS = 8.0
_NEG = -1e30


def _rotl(x, r):
    return (lax.shift_left(x, jnp.uint32(r))
            | lax.shift_right_logical(x, jnp.uint32(32 - r)))


def _threefry_bits(k1, k2, idx):
    ks2 = k1 ^ k2 ^ jnp.uint32(0x1BD11BDA)
    x0 = jnp.broadcast_to(k1, idx.shape)
    x1 = idx + k2
    rot = ((13, 15, 26, 6), (17, 29, 16, 24))
    inj = ((k2, ks2, 1), (ks2, k1, 2), (k1, k2, 3), (k2, ks2, 4), (ks2, k1, 5))
    for g in range(5):
        for r in rot[g % 2]:
            x0 = x0 + x1
            x1 = _rotl(x1, r)
            x1 = x0 ^ x1
        a, bq, c = inj[g]
        x0 = x0 + a
        x1 = x1 + (bq + jnp.uint32(c))
    return x0 ^ x1


def _vat_kernel(key_ref, x_ref, w_ref, b_ref, lab_ref, out_ref, gram_ref,
                *, grad_scale, tile, feat):
    i = pl.program_id(0)
    x = x_ref[...]
    w = w_ref[...]
    b = b_ref[...]
    lab = lab_ref[...]

    @pl.when(i == 0)
    def _():
        gram_ref[...] = lax.dot_general(w, w, (((0,), (0,)), ((), ())),
                                        preferred_element_type=jnp.float32)
    gram = gram_ref[...]

    k1 = key_ref[0]
    k2 = key_ref[1]
    row = lax.broadcasted_iota(jnp.uint32, (tile, feat), 0)
    col = lax.broadcasted_iota(jnp.uint32, (tile, feat), 1)
    base = (jnp.uint32(i) * jnp.uint32(tile)) * jnp.uint32(feat)
    idx = base + row * jnp.uint32(feat) + col
    bits = _threefry_bits(k1, k2, idx)
    fbits = lax.bitcast_convert_type(
        lax.shift_right_logical(bits, jnp.uint32(9)) | jnp.uint32(0x3F800000),
        jnp.float32)
    d0 = fbits - 1.5

    ss0 = jnp.sum(d0 * d0, axis=1, keepdims=True)
    r0 = lax.rsqrt(jnp.maximum(ss0, 1e-16))

    lc = jnp.dot(x.astype(jnp.bfloat16), w,
                 preferred_element_type=jnp.float32) + b
    u = jnp.dot(d0.astype(jnp.bfloat16), w,
                preferred_element_type=jnp.float32)
    lh1 = lc + _XI * (u * r0)

    m0 = jnp.max(lc, axis=1, keepdims=True)
    e0 = jnp.exp(lc - m0)
    s0 = jnp.sum(e0, axis=1, keepdims=True)
    log_s0 = jnp.log(s0)
    p = e0 * (1.0 / s0)
    logp = (lc - m0) - log_s0
    lse = m0 + log_s0

    m1 = jnp.max(lh1, axis=1, keepdims=True)
    e1 = jnp.exp(lh1 - m1)
    q1 = e1 * (1.0 / jnp.sum(e1, axis=1, keepdims=True))
    delta = (q1 - p) * grad_scale
    v = jnp.dot(delta.astype(jnp.bfloat16), gram.astype(jnp.bfloat16),
                preferred_element_type=jnp.float32)
    ssg = jnp.sum(delta * v, axis=1, keepdims=True)
    r1 = lax.rsqrt(jnp.maximum(ssg, 1e-16))

    lh = lc + _EPS * (v * r1)
    m2 = jnp.max(lh, axis=1, keepdims=True)
    e2 = jnp.exp(lh - m2)
    logq = (lh - m2) - jnp.log(jnp.sum(e2, axis=1, keepdims=True))
    kl_row = jnp.sum(p * (logp - logq), axis=1, keepdims=True)

    col_i = lax.broadcasted_iota(jnp.int32, lc.shape, 1)
    xy_row = jnp.sum(jnp.where(col_i == lab, lc, 0.0), axis=1, keepdims=True)

    kl_s = jnp.sum(kl_row)
    lse_s = jnp.sum(lse)
    xy_s = jnp.sum(xy_row)
    lane = lax.broadcasted_iota(jnp.int32, (1, 1, 128), 2)
    out_ref[...] = (jnp.where(lane == 0, kl_s, 0.0)
                    + jnp.where(lane == 1, lse_s, 0.0)
                    + jnp.where(lane == 2, xy_s, 0.0))


def kernel(w, b, x, labels, d_key):
    n = x.shape[0]
    xf = x.reshape(n, -1).astype(jnp.float32)
    f = xf.shape[1]
    c = w.shape[1]
    cp = ((c + 127) // 128) * 128

    w_p = jnp.pad(w.astype(jnp.float32), ((0, 0), (0, cp - c))).astype(jnp.bfloat16)
    b_p = jnp.pad(b.astype(jnp.float32)[None, :], ((0, 0), (0, cp - c)),
                  constant_values=_NEG)
    lab2 = labels.astype(jnp.int32)[:, None]
    key_words = d_key.reshape(2).astype(jnp.uint32)

    tile = 512 if n % 512 == 0 else n
    nb = n // tile

    fn = functools.partial(_vat_kernel, grad_scale=_XI / n, tile=tile, feat=f)
    out = pl.pallas_call(
        fn,
        out_shape=jax.ShapeDtypeStruct((nb, 1, 128), jnp.float32),
        grid=(nb,),
        in_specs=[
            pl.BlockSpec(memory_space=pltpu.SMEM),
            pl.BlockSpec((tile, f), lambda i: (i, 0)),
            pl.BlockSpec((f, cp), lambda i: (0, 0)),
            pl.BlockSpec((1, cp), lambda i: (0, 0)),
            pl.BlockSpec((tile, 1), lambda i: (i, 0)),
        ],
        out_specs=pl.BlockSpec((1, 1, 128), lambda i: (i, 0, 0)),
        scratch_shapes=[pltpu.VMEM((cp, cp), jnp.float32)],
        compiler_params=pltpu.CompilerParams(
            dimension_semantics=("arbitrary",),
            vmem_limit_bytes=48 * 1024 * 1024,
        ),
    )(key_words, xf, w_p, b_p, lab2)

    parts = out.reshape(nb, 128)
    kl_sum = jnp.sum(parts[:, 0])
    lse_sum = jnp.sum(parts[:, 1])
    xy_sum = jnp.sum(parts[:, 2])
    return (lse_sum - xy_sum) / n + kl_sum / n


# speedup vs baseline: 1.0331x; 1.0331x over previous
import functools

import jax
import jax.numpy as jnp
from jax import lax
from jax.experimental import pallas as pl
from jax.experimental.pallas import tpu as pltpu

_XI = 10.0
_EPS = 8.0
_NEG = -1e30


def _gram_kernel(w_ref, g_ref):
    w = w_ref[...]
    g_ref[...] = lax.dot_general(w, w, (((0,), (0,)), ((), ())),
                                 preferred_element_type=jnp.float32)


def _rotl(x, r):
    return (lax.shift_left(x, jnp.uint32(r))
            | lax.shift_right_logical(x, jnp.uint32(32 - r)))


def _threefry_bits(k1, k2, idx):
    ks2 = k1 ^ k2 ^ jnp.uint32(0x1BD11BDA)
    x0 = jnp.broadcast_to(k1, idx.shape)
    x1 = idx + k2
    rot = ((13, 15, 26, 6), (17, 29, 16, 24))
    inj = ((k2, ks2, 1), (ks2, k1, 2), (k1, k2, 3), (k2, ks2, 4), (ks2, k1, 5))
    for g in range(5):
        for r in rot[g % 2]:
            x0 = x0 + x1
            x1 = _rotl(x1, r)
            x1 = x0 ^ x1
        a, bq, c = inj[g]
        x0 = x0 + a
        x1 = x1 + (bq + jnp.uint32(c))
    return x0 ^ x1


def _vat_kernel(key_ref, x_ref, w_ref, gram_ref, b_ref, lab_ref, out_ref,
                acc_ref, *, grad_scale, tile, feat, inv_n):
    i = pl.program_id(0)
    x = x_ref[...]
    w = w_ref[...]
    gram = gram_ref[...]
    b = b_ref[...]
    lab = lab_ref[...]

    k1 = key_ref[0]
    k2 = key_ref[1]
    row = lax.broadcasted_iota(jnp.uint32, (tile, feat), 0)
    col = lax.broadcasted_iota(jnp.uint32, (tile, feat), 1)
    base = (jnp.uint32(i) * jnp.uint32(tile)) * jnp.uint32(feat)
    idx = base + row * jnp.uint32(feat) + col
    bits = _threefry_bits(k1, k2, idx)
    fbits = lax.bitcast_convert_type(
        lax.shift_right_logical(bits, jnp.uint32(9)) | jnp.uint32(0x3F800000),
        jnp.float32)
    d0 = fbits - 1.5

    ss0 = jnp.sum(d0 * d0, axis=1, keepdims=True)
    r0 = lax.rsqrt(jnp.maximum(ss0, 1e-16))

    lc = jnp.dot(x.astype(jnp.bfloat16), w,
                 preferred_element_type=jnp.float32) + b
    u = jnp.dot(d0.astype(jnp.bfloat16), w,
                preferred_element_type=jnp.float32)
    lh1 = lc + _XI * (u * r0)

    m0 = jnp.max(lc, axis=1, keepdims=True)
    e0 = jnp.exp(lc - m0)
    s0 = jnp.sum(e0, axis=1, keepdims=True)
    log_s0 = jnp.log(s0)
    p = e0 * (1.0 / s0)
    logp = (lc - m0) - log_s0
    lse = m0 + log_s0

    m1 = jnp.max(lh1, axis=1, keepdims=True)
    e1 = jnp.exp(lh1 - m1)
    q1 = e1 * (1.0 / jnp.sum(e1, axis=1, keepdims=True))
    delta = (q1 - p) * grad_scale
    v = jnp.dot(delta.astype(jnp.bfloat16), gram.astype(jnp.bfloat16),
                preferred_element_type=jnp.float32)
    ssg = jnp.sum(delta * v, axis=1, keepdims=True)
    r1 = lax.rsqrt(jnp.maximum(ssg, 1e-16))

    lh = lc + _EPS * (v * r1)
    m2 = jnp.max(lh, axis=1, keepdims=True)
    e2 = jnp.exp(lh - m2)
    logq = (lh - m2) - jnp.log(jnp.sum(e2, axis=1, keepdims=True))
    kl_row = jnp.sum(p * (logp - logq), axis=1, keepdims=True)

    col_i = lax.broadcasted_iota(jnp.int32, lc.shape, 1)
    xy_row = jnp.sum(jnp.where(col_i == lab, lc, 0.0), axis=1, keepdims=True)

    step_sum = jnp.sum(kl_row + lse - xy_row)

    @pl.when(i == 0)
    def _():
        acc_ref[...] = jnp.zeros_like(acc_ref)

    acc_ref[...] += step_sum

    @pl.when(i == pl.num_programs(0) - 1)
    def _():
        out_ref[...] = acc_ref[...] * inv_n


def kernel(w, b, x, labels, d_key):
    n = x.shape[0]
    xf = x.reshape(n, -1).astype(jnp.float32)
    f = xf.shape[1]
    c = w.shape[1]
    cp = ((c + 127) // 128) * 128

    w_p = jnp.pad(w.astype(jnp.float32), ((0, 0), (0, cp - c))).astype(jnp.bfloat16)
    b_p = jnp.pad(b.astype(jnp.float32)[None, :], ((0, 0), (0, cp - c)),
                  constant_values=_NEG)
    lab2 = labels.astype(jnp.int32)[:, None]
    key_words = d_key.reshape(2).astype(jnp.uint32)

    gram = pl.pallas_call(
        _gram_kernel,
        out_shape=jax.ShapeDtypeStruct((cp, cp), jnp.float32),
        compiler_params=pltpu.CompilerParams(
            vmem_limit_bytes=48 * 1024 * 1024,
        ),
    )(w_p)

    tile = 512 if n % 512 == 0 else n
    nb = n // tile

    fn = functools.partial(_vat_kernel, grad_scale=_XI / n, tile=tile, feat=f,
                           inv_n=1.0 / n)
    out = pl.pallas_call(
        fn,
        out_shape=jax.ShapeDtypeStruct((1, 1), jnp.float32),
        grid=(nb,),
        in_specs=[
            pl.BlockSpec(memory_space=pltpu.SMEM),
            pl.BlockSpec((tile, f), lambda i: (i, 0)),
            pl.BlockSpec((f, cp), lambda i: (0, 0)),
            pl.BlockSpec((cp, cp), lambda i: (0, 0)),
            pl.BlockSpec((1, cp), lambda i: (0, 0)),
            pl.BlockSpec((tile, 1), lambda i: (i, 0)),
        ],
        out_specs=pl.BlockSpec((1, 1), lambda i: (0, 0)),
        scratch_shapes=[pltpu.VMEM((1, 1), jnp.float32)],
        compiler_params=pltpu.CompilerParams(
            dimension_semantics=("arbitrary",),
            vmem_limit_bytes=48 * 1024 * 1024,
        ),
    )(key_words, xf, w_p, gram, b_p, lab2)

    return out.reshape(())


# speedup vs baseline: 1.0394x; 1.0060x over previous
import functools

import jax
import jax.numpy as jnp
from jax import lax
from jax.experimental import pallas as pl
from jax.experimental.pallas import tpu as pltpu

_XI = 10.0
_EPS = 8.0
_NEG = -1e30


def _gram_kernel(w_ref, g_ref):
    w = w_ref[...]
    g_ref[...] = lax.dot_general(w, w, (((0,), (0,)), ((), ())),
                                 preferred_element_type=jnp.float32)


def _rotl(x, r):
    return (lax.shift_left(x, jnp.uint32(r))
            | lax.shift_right_logical(x, jnp.uint32(32 - r)))


def _threefry_bits(k1, k2, idx):
    ks2 = k1 ^ k2 ^ jnp.uint32(0x1BD11BDA)
    x0 = jnp.broadcast_to(k1, idx.shape)
    x1 = idx + k2
    rot = ((13, 15, 26, 6), (17, 29, 16, 24))
    inj = ((k2, ks2, 1), (ks2, k1, 2), (k1, k2, 3), (k2, ks2, 4), (ks2, k1, 5))
    for g in range(5):
        for r in rot[g % 2]:
            x0 = x0 + x1
            x1 = _rotl(x1, r)
            x1 = x0 ^ x1
        a, bq, c = inj[g]
        x0 = x0 + a
        x1 = x1 + (bq + jnp.uint32(c))
    return x0 ^ x1


def _vat_kernel(key_ref, x_ref, w_ref, gram_ref, b_ref, lab_ref, out_ref,
                acc_ref, *, grad_scale, tile, feat, inv_n):
    i = pl.program_id(0)
    x = x_ref[...]
    w = w_ref[...]
    gram = gram_ref[...]
    b = b_ref[...]
    lab = lab_ref[...]

    k1 = key_ref[0]
    k2 = key_ref[1]
    row = lax.broadcasted_iota(jnp.uint32, (tile, feat), 0)
    col = lax.broadcasted_iota(jnp.uint32, (tile, feat), 1)
    base = (jnp.uint32(i) * jnp.uint32(tile)) * jnp.uint32(feat)
    idx = base + row * jnp.uint32(feat) + col
    bits = _threefry_bits(k1, k2, idx)
    fbits = lax.bitcast_convert_type(
        lax.shift_right_logical(bits, jnp.uint32(9)) | jnp.uint32(0x3F800000),
        jnp.float32)
    d0 = fbits - 1.5

    ss0 = jnp.sum(d0 * d0, axis=1, keepdims=True)
    r0 = lax.rsqrt(jnp.maximum(ss0, 1e-16))

    lc = jnp.dot(x.astype(jnp.bfloat16), w,
                 preferred_element_type=jnp.float32) + b
    u = jnp.dot(d0.astype(jnp.bfloat16), w,
                preferred_element_type=jnp.float32)
    lh1 = lc + _XI * (u * r0)

    m0 = jnp.max(lc, axis=1, keepdims=True)
    e0 = jnp.exp(lc - m0)
    s0 = jnp.sum(e0, axis=1, keepdims=True)
    log_s0 = jnp.log(s0)
    p = e0 * (1.0 / s0)
    logp = (lc - m0) - log_s0
    lse = m0 + log_s0

    m1 = jnp.max(lh1, axis=1, keepdims=True)
    e1 = jnp.exp(lh1 - m1)
    q1 = e1 * (1.0 / jnp.sum(e1, axis=1, keepdims=True))
    delta = (q1 - p) * grad_scale
    v = jnp.dot(delta.astype(jnp.bfloat16), gram.astype(jnp.bfloat16),
                preferred_element_type=jnp.float32)
    ssg = jnp.sum(delta * v, axis=1, keepdims=True)
    r1 = lax.rsqrt(jnp.maximum(ssg, 1e-16))

    lh = lc + _EPS * (v * r1)
    m2 = jnp.max(lh, axis=1, keepdims=True)
    e2 = jnp.exp(lh - m2)
    logq = (lh - m2) - jnp.log(jnp.sum(e2, axis=1, keepdims=True))
    kl_row = jnp.sum(p * (logp - logq), axis=1, keepdims=True)

    col_i = lax.broadcasted_iota(jnp.int32, lc.shape, 1)
    xy_row = jnp.sum(jnp.where(col_i == lab, lc, 0.0), axis=1, keepdims=True)

    step_sum = jnp.sum(kl_row + lse - xy_row)

    @pl.when(i == 0)
    def _():
        acc_ref[...] = jnp.zeros_like(acc_ref)

    acc_ref[...] += step_sum

    @pl.when(i == pl.num_programs(0) - 1)
    def _():
        out_ref[...] = acc_ref[...] * inv_n


def kernel(w, b, x, labels, d_key):
    n = x.shape[0]
    xf = x.reshape(n, -1).astype(jnp.float32)
    f = xf.shape[1]
    c = w.shape[1]

    w_b = w.astype(jnp.bfloat16)
    b_2 = b.astype(jnp.float32)[None, :]
    lab2 = labels.astype(jnp.int32)[:, None]
    key_words = d_key.reshape(2).astype(jnp.uint32)

    gram = pl.pallas_call(
        _gram_kernel,
        out_shape=jax.ShapeDtypeStruct((c, c), jnp.float32),
        compiler_params=pltpu.CompilerParams(
            vmem_limit_bytes=48 * 1024 * 1024,
        ),
    )(w_b)

    tile = 512 if n % 512 == 0 else n
    nb = n // tile

    fn = functools.partial(_vat_kernel, grad_scale=_XI / n, tile=tile, feat=f,
                           inv_n=1.0 / n)
    out = pl.pallas_call(
        fn,
        out_shape=jax.ShapeDtypeStruct((1, 1), jnp.float32),
        grid=(nb,),
        in_specs=[
            pl.BlockSpec(memory_space=pltpu.SMEM),
            pl.BlockSpec((tile, f), lambda i: (i, 0)),
            pl.BlockSpec((f, c), lambda i: (0, 0)),
            pl.BlockSpec((c, c), lambda i: (0, 0)),
            pl.BlockSpec((1, c), lambda i: (0, 0)),
            pl.BlockSpec((tile, 1), lambda i: (i, 0)),
        ],
        out_specs=pl.BlockSpec((1, 1), lambda i: (0, 0)),
        scratch_shapes=[pltpu.VMEM((1, 1), jnp.float32)],
        compiler_params=pltpu.CompilerParams(
            dimension_semantics=("arbitrary",),
            vmem_limit_bytes=48 * 1024 * 1024,
        ),
    )(key_words, xf, w_b, gram, b_2, lab2)

    return out.reshape(())


# speedup vs baseline: 1.1705x; 1.1262x over previous
import functools

import jax
import jax.numpy as jnp
from jax import lax
from jax.experimental import pallas as pl
from jax.experimental.pallas import tpu as pltpu

_XI = 10.0
_EPS = 8.0
_NEG = -1e30


def _gram_kernel(w_ref, g_ref):
    w = w_ref[...]
    g_ref[...] = lax.dot_general(w, w, (((0,), (0,)), ((), ())),
                                 preferred_element_type=jnp.float32)


def _rotl(x, r):
    return (lax.shift_left(x, jnp.uint32(r))
            | lax.shift_right_logical(x, jnp.uint32(32 - r)))


def _threefry_bits(k1, k2, idx):
    ks2 = k1 ^ k2 ^ jnp.uint32(0x1BD11BDA)
    x0 = jnp.broadcast_to(k1, idx.shape)
    x1 = idx + k2
    rot = ((13, 15, 26, 6), (17, 29, 16, 24))
    inj = ((k2, ks2, 1), (ks2, k1, 2), (k1, k2, 3), (k2, ks2, 4), (ks2, k1, 5))
    for g in range(5):
        for r in rot[g % 2]:
            x0 = x0 + x1
            x1 = _rotl(x1, r)
            x1 = x0 ^ x1
        a, bq, c = inj[g]
        x0 = x0 + a
        x1 = x1 + (bq + jnp.uint32(c))
    return x0 ^ x1


def _vat_kernel(key_ref, lc_ref, w_ref, gram_ref, lab_ref, out_ref,
                acc_ref, *, grad_scale, tile, feat, inv_n):
    i = pl.program_id(0)
    lc = lc_ref[...]
    w = w_ref[...]
    gram = gram_ref[...]
    lab = lab_ref[...]

    k1 = key_ref[0]
    k2 = key_ref[1]
    row = lax.broadcasted_iota(jnp.uint32, (tile, feat), 0)
    col = lax.broadcasted_iota(jnp.uint32, (tile, feat), 1)
    base = (jnp.uint32(i) * jnp.uint32(tile)) * jnp.uint32(feat)
    idx = base + row * jnp.uint32(feat) + col
    bits = _threefry_bits(k1, k2, idx)
    fbits = lax.bitcast_convert_type(
        lax.shift_right_logical(bits, jnp.uint32(9)) | jnp.uint32(0x3F800000),
        jnp.float32)
    d0 = fbits - 1.5

    ss0 = jnp.sum(d0 * d0, axis=1, keepdims=True)
    r0 = lax.rsqrt(jnp.maximum(ss0, 1e-16))

    u = jnp.dot(d0.astype(jnp.bfloat16), w,
                preferred_element_type=jnp.float32)
    lh1 = lc + _XI * (u * r0)

    m0 = jnp.max(lc, axis=1, keepdims=True)
    e0 = jnp.exp(lc - m0)
    s0 = jnp.sum(e0, axis=1, keepdims=True)
    log_s0 = jnp.log(s0)
    p = e0 * (1.0 / s0)
    logp = (lc - m0) - log_s0
    lse = m0 + log_s0

    m1 = jnp.max(lh1, axis=1, keepdims=True)
    e1 = jnp.exp(lh1 - m1)
    q1 = e1 * (1.0 / jnp.sum(e1, axis=1, keepdims=True))
    delta = (q1 - p) * grad_scale
    v = jnp.dot(delta.astype(jnp.bfloat16), gram.astype(jnp.bfloat16),
                preferred_element_type=jnp.float32)
    ssg = jnp.sum(delta * v, axis=1, keepdims=True)
    r1 = lax.rsqrt(jnp.maximum(ssg, 1e-16))

    lh = lc + _EPS * (v * r1)
    m2 = jnp.max(lh, axis=1, keepdims=True)
    e2 = jnp.exp(lh - m2)
    logq = (lh - m2) - jnp.log(jnp.sum(e2, axis=1, keepdims=True))
    kl_row = jnp.sum(p * (logp - logq), axis=1, keepdims=True)

    col_i = lax.broadcasted_iota(jnp.int32, lc.shape, 1)
    xy_row = jnp.sum(jnp.where(col_i == lab, lc, 0.0), axis=1, keepdims=True)

    step_sum = jnp.sum(kl_row + lse - xy_row)

    @pl.when(i == 0)
    def _():
        acc_ref[...] = jnp.zeros_like(acc_ref)

    acc_ref[...] += step_sum

    @pl.when(i == pl.num_programs(0) - 1)
    def _():
        out_ref[...] = acc_ref[...] * inv_n


def kernel(w, b, x, labels, d_key):
    n = x.shape[0]
    f = w.shape[0]
    c = w.shape[1]

    lc_clean = jnp.dot(x.reshape(n, f), w, preferred_element_type=jnp.float32) + b

    w_b = w.astype(jnp.bfloat16)
    lab2 = labels.astype(jnp.int32)[:, None]
    key_words = d_key.reshape(2).astype(jnp.uint32)

    gram = pl.pallas_call(
        _gram_kernel,
        out_shape=jax.ShapeDtypeStruct((c, c), jnp.float32),
        compiler_params=pltpu.CompilerParams(
            vmem_limit_bytes=48 * 1024 * 1024,
        ),
    )(w_b)

    tile = 512 if n % 512 == 0 else n
    nb = n // tile

    fn = functools.partial(_vat_kernel, grad_scale=_XI / n, tile=tile, feat=f,
                           inv_n=1.0 / n)
    out = pl.pallas_call(
        fn,
        out_shape=jax.ShapeDtypeStruct((1, 1), jnp.float32),
        grid=(nb,),
        in_specs=[
            pl.BlockSpec(memory_space=pltpu.SMEM),
            pl.BlockSpec((tile, c), lambda i: (i, 0)),
            pl.BlockSpec((f, c), lambda i: (0, 0)),
            pl.BlockSpec((c, c), lambda i: (0, 0)),
            pl.BlockSpec((tile, 1), lambda i: (i, 0)),
        ],
        out_specs=pl.BlockSpec((1, 1), lambda i: (0, 0)),
        scratch_shapes=[pltpu.VMEM((1, 1), jnp.float32)],
        compiler_params=pltpu.CompilerParams(
            dimension_semantics=("arbitrary",),
            vmem_limit_bytes=48 * 1024 * 1024,
        ),
    )(key_words, lc_clean, w_b, gram, lab2)

    return out.reshape(())


# speedup vs baseline: 1.1708x; 1.0002x over previous
import functools

import jax
import jax.numpy as jnp
from jax import lax
from jax.experimental import pallas as pl
from jax.experimental.pallas import tpu as pltpu

_XI = 10.0
_EPS = 8.0


def _gram_kernel(w_ref, g_ref):
    w = w_ref[...]
    g_ref[...] = lax.dot_general(w, w, (((0,), (0,)), ((), ())),
                                 preferred_element_type=jnp.float32)


def _rotl(x, r):
    return (lax.shift_left(x, jnp.uint32(r))
            | lax.shift_right_logical(x, jnp.uint32(32 - r)))


def _threefry_bits(k1, k2, idx):
    ks2 = k1 ^ k2 ^ jnp.uint32(0x1BD11BDA)
    x0 = jnp.broadcast_to(k1, idx.shape)
    x1 = idx + k2
    rot = ((13, 15, 26, 6), (17, 29, 16, 24))
    inj = ((k2, ks2, 1), (ks2, k1, 2), (k1, k2, 3), (k2, ks2, 4), (ks2, k1, 5))
    for g in range(5):
        for r in rot[g % 2]:
            x0 = x0 + x1
            x1 = _rotl(x1, r)
            x1 = x0 ^ x1
        a, bq, c = inj[g]
        x0 = x0 + a
        x1 = x1 + (bq + jnp.uint32(c))
    return x0 ^ x1


def _vat_kernel(key_ref, lc_ref, w_ref, gram_ref, lab_ref, out_ref,
                acc_ref, *, grad_scale, tile, feat, inv_n):
    i = pl.program_id(0)
    lc = lc_ref[...]
    w = w_ref[...]
    gram = gram_ref[...]
    lab = lab_ref[...]

    k1 = key_ref[0]
    k2 = key_ref[1]
    row = lax.broadcasted_iota(jnp.uint32, (tile, feat), 0)
    col = lax.broadcasted_iota(jnp.uint32, (tile, feat), 1)
    base = (jnp.uint32(i) * jnp.uint32(tile)) * jnp.uint32(feat)
    idx = base + row * jnp.uint32(feat) + col
    bits = _threefry_bits(k1, k2, idx)
    fbits = lax.bitcast_convert_type(
        lax.shift_right_logical(bits, jnp.uint32(9)) | jnp.uint32(0x3F800000),
        jnp.float32)
    d0 = fbits - 1.5

    ss0 = jnp.sum(d0 * d0, axis=1, keepdims=True)
    r0 = lax.rsqrt(jnp.maximum(ss0, 1e-16))

    u = jnp.dot(d0.astype(jnp.bfloat16), w,
                preferred_element_type=jnp.float32)
    lh1 = lc + _XI * (u * r0)

    m0 = jnp.max(lc, axis=1, keepdims=True)
    e0 = jnp.exp(lc - m0)
    s0 = jnp.sum(e0, axis=1, keepdims=True)
    log_s0 = jnp.log(s0)
    p = e0 * (1.0 / s0)
    logp = (lc - m0) - log_s0
    lse = m0 + log_s0

    m1 = jnp.max(lh1, axis=1, keepdims=True)
    e1 = jnp.exp(lh1 - m1)
    q1 = e1 * (1.0 / jnp.sum(e1, axis=1, keepdims=True))
    delta = (q1 - p) * grad_scale
    v = jnp.dot(delta.astype(jnp.bfloat16), gram.astype(jnp.bfloat16),
                preferred_element_type=jnp.float32)
    ssg = jnp.sum(delta * v, axis=1, keepdims=True)
    r1 = lax.rsqrt(jnp.maximum(ssg, 1e-16))

    lh = lc + _EPS * (v * r1)
    m2 = jnp.max(lh, axis=1, keepdims=True)
    e2 = jnp.exp(lh - m2)
    logq = (lh - m2) - jnp.log(jnp.sum(e2, axis=1, keepdims=True))
    kl_row = jnp.sum(p * (logp - logq), axis=1, keepdims=True)

    col_i = lax.broadcasted_iota(jnp.int32, lc.shape, 1)
    xy_row = jnp.sum(jnp.where(col_i == lab, lc, 0.0), axis=1, keepdims=True)

    step_sum = jnp.sum(kl_row + lse - xy_row)

    @pl.when(i == 0)
    def _():
        acc_ref[...] = jnp.zeros_like(acc_ref)

    acc_ref[...] += step_sum

    @pl.when(i == pl.num_programs(0) - 1)
    def _():
        out_ref[...] = acc_ref[...] * inv_n


def kernel(w, b, x, labels, d_key):
    n = x.shape[0]
    f = w.shape[0]
    c = w.shape[1]

    lc_clean = jnp.dot(x.reshape(n, f), w, preferred_element_type=jnp.float32) + b

    w_b = w.astype(jnp.bfloat16)
    lab2 = labels.astype(jnp.int32)[:, None]
    key_words = d_key.reshape(2).astype(jnp.uint32)

    gram = pl.pallas_call(
        _gram_kernel,
        out_shape=jax.ShapeDtypeStruct((c, c), jnp.float32),
        compiler_params=pltpu.CompilerParams(
            vmem_limit_bytes=48 * 1024 * 1024,
        ),
    )(w_b)

    tile = 512 if n % 512 == 0 else n
    nb = n // tile

    fn = functools.partial(_vat_kernel, grad_scale=_XI / n, tile=tile, feat=f,
                           inv_n=1.0 / n)
    out = pl.pallas_call(
        fn,
        out_shape=jax.ShapeDtypeStruct((1, 1), jnp.float32),
        grid=(nb,),
        in_specs=[
            pl.BlockSpec(memory_space=pltpu.SMEM),
            pl.BlockSpec((tile, c), lambda i: (i, 0)),
            pl.BlockSpec((f, c), lambda i: (0, 0)),
            pl.BlockSpec((c, c), lambda i: (0, 0)),
            pl.BlockSpec((tile, 1), lambda i: (i, 0)),
        ],
        out_specs=pl.BlockSpec((1, 1), lambda i: (0, 0)),
        scratch_shapes=[pltpu.VMEM((1, 1), jnp.float32)],
        compiler_params=pltpu.CompilerParams(
            dimension_semantics=("arbitrary",),
            vmem_limit_bytes=48 * 1024 * 1024,
        ),
    )(key_words, lc_clean, w_b, gram, lab2)

    return out.reshape(())
